# baseline (speedup 1.0000x reference)
import functools

import jax
import jax.numpy as jnp
from jax import lax
from jax.experimental import pallas as pl
from jax.experimental.pallas import tpu as pltpu
from jax.experimental.pallas import tpu_sc as plsc

_NS = 16
_NC = 2
_NW = _NC * _NS
_CHUNK = 64
_W = 144


def _tc_message_ffn(x_ref, w_ref, b_ref, o_ref):
    t = jnp.tanh(
        jnp.dot(x_ref[...], w_ref[...], preferred_element_type=jnp.float32)
        + b_ref[...]
    )
    ones = jnp.ones((t.shape[0], _W - t.shape[1]), jnp.float32)
    o_ref[...] = jnp.concatenate([t, ones], axis=1)


def _tc_update_ffn(x_ref, p_ref, w2a_ref, w2b_ref, b_ref, o_ref):
    ssum = p_ref[0] + p_ref[1]
    cnt = jnp.maximum(ssum[:, 128:129], 1.0)
    agg = ssum[:, :128] / cnt
    o_ref[...] = jnp.tanh(
        jnp.dot(x_ref[...], w2a_ref[...], preferred_element_type=jnp.float32)
        + jnp.dot(agg, w2b_ref[...], preferred_element_type=jnp.float32)
        + b_ref[...]
    )


def _make_sc_aggregate(np_, chmax, chs0, chs1):
    stripe = np_ // _NS
    mesh = plsc.VectorSubcoreMesh(core_axis_name="c", subcore_axis_name="s")
    depth = 2

    @functools.partial(
        pl.kernel,
        mesh=mesh,
        out_type=jax.ShapeDtypeStruct((_NC, np_, _W), jnp.float32),
        scratch_types=[
            pltpu.VMEM((chmax, _CHUNK), jnp.int32),
            pltpu.VMEM((depth, _CHUNK), jnp.int32),
            pltpu.VMEM((depth, _CHUNK), jnp.int32),
            [pltpu.VMEM((_CHUNK, _W), jnp.float32) for _ in range(depth)],
            pltpu.VMEM_SHARED((np_, _W), jnp.float32),
            [pltpu.SemaphoreType.DMA for _ in range(depth)],
        ],
        compiler_params=pltpu.CompilerParams(use_tc_tiling_on_sc=False),
    )
    def sc_aggregate(yext, packed, zeros, out, packed_v, nbr_v, dst_v, rows, acc, sems):
        c = lax.axis_index("c")
        s = lax.axis_index("s")
        w = c * _NS + s
        row0 = s * stripe
        my_ch = jnp.where(c == 0, chs0, chs1)
        pltpu.sync_copy(zeros.at[pl.ds(row0, stripe)], acc.at[pl.ds(row0, stripe)])
        pltpu.sync_copy(packed.at[w], packed_v)
        plsc.subcore_barrier()

        def unpack(j, slot):
            for i in range(_CHUNK // 16):
                v = packed_v[j, pl.ds(16 * i, 16)]
                nbr_v[slot, pl.ds(16 * i, 16)] = lax.bitwise_and(v, 0xFFFF)
                dst_v[slot, pl.ds(16 * i, 16)] = lax.shift_right_logical(v, 16)

        def fire(j, slot):
            unpack(j, slot)
            pltpu.async_copy(yext.at[nbr_v.at[slot]], rows[slot], sems[slot])

        for k in range(depth - 1):
            fire(k, k)

        def step(j, slot):
            nxt = (slot + depth - 1) % depth

            @pl.when(j + depth - 1 < my_ch)
            def _():
                fire(j + depth - 1, nxt)

            @pl.when(j < my_ch)
            def _():
                pltpu.make_async_copy(
                    yext.at[nbr_v.at[slot]], rows[slot], sems[slot]
                ).wait()
                pltpu.sync_copy(rows[slot], acc.at[dst_v.at[slot]], add=True)

        def body(q, carry):
            j0 = depth * q
            for k in range(depth):
                step(j0 + k, k)
            return carry

        lax.fori_loop(0, chmax // depth, body, 0)
        plsc.subcore_barrier()
        pltpu.sync_copy(
            acc.at[pl.ds(row0, stripe)], out.at[c, pl.ds(row0, stripe)]
        )

    return sc_aggregate


def kernel(inputs, edges, edge_weights, W1, b1, W2, b2):
    del edge_weights
    _, n, d = inputs.shape
    h = W1.shape[1]
    e = edges.shape[1]

    np_ = ((n + _NS * 8 - 1) // (_NS * 8)) * (_NS * 8)
    frac0 = 0.58
    ct = -(-e // _CHUNK)
    chs0 = max(2, (int(ct * frac0) // (_NS * 2)) * 2)
    chs1 = -(-(ct - _NS * chs0) // _NS)
    chs1 += chs1 % 2
    chmax = max(chs0, chs1)
    ep = _NS * (chs0 + chs1) * _CHUNK

    x = inputs[0]
    xp = jnp.pad(x, ((0, np_ - n), (0, 0)))
    nbr = jnp.pad(edges[1], (0, ep - e))
    dst = jnp.pad(edges[0], (0, ep - e), constant_values=n)
    packed_flat = nbr | (dst << 16)
    pad_val = n << 16
    split = _NS * chs0 * _CHUNK
    part0 = packed_flat[:split].reshape(_NS, chs0, _CHUNK)
    part1 = packed_flat[split:].reshape(_NS, chs1, _CHUNK)
    part0 = jnp.pad(part0, ((0, 0), (0, chmax - chs0), (0, 0)),
                    constant_values=pad_val)
    part1 = jnp.pad(part1, ((0, 0), (0, chmax - chs1), (0, 0)),
                    constant_values=pad_val)
    packed = jnp.concatenate([part0, part1], axis=0)
    zeros = jnp.zeros((np_, _W), jnp.float32)

    yext = pl.pallas_call(
        _tc_message_ffn,
        out_shape=jax.ShapeDtypeStruct((np_, _W), jnp.float32),
    )(xp, W1, b1.reshape(1, h))

    partials = _make_sc_aggregate(np_, chmax, chs0, chs1)(yext, packed, zeros)

    out = pl.pallas_call(
        _tc_update_ffn,
        out_shape=jax.ShapeDtypeStruct((np_, h), jnp.float32),
    )(xp, partials, W2[:d], W2[d:], b2.reshape(1, h))

    return out[:n][None]


# speedup vs baseline: 1.0055x; 1.0055x over previous
import functools

import jax
import jax.numpy as jnp
from jax import lax
from jax.experimental import pallas as pl
from jax.experimental.pallas import tpu as pltpu
from jax.experimental.pallas import tpu_sc as plsc

_NS = 16
_NC = 2
_NW = _NC * _NS
_CHUNK = 64
_W = 144


def _tc_message_ffn(x_ref, w_ref, b_ref, o_ref):
    t = jnp.tanh(
        jnp.dot(x_ref[...], w_ref[...], preferred_element_type=jnp.float32)
        + b_ref[...]
    )
    ones = jnp.ones((t.shape[0], _W - t.shape[1]), jnp.float32)
    o_ref[...] = jnp.concatenate([t, ones], axis=1)


def _tc_update_ffn(x_ref, p_ref, w2a_ref, w2b_ref, b_ref, o_ref):
    ssum = p_ref[0] + p_ref[1]
    cnt = jnp.maximum(ssum[:, 128:129], 1.0)
    agg = ssum[:, :128] / cnt
    o_ref[...] = jnp.tanh(
        jnp.dot(x_ref[...], w2a_ref[...], preferred_element_type=jnp.float32)
        + jnp.dot(agg, w2b_ref[...], preferred_element_type=jnp.float32)
        + b_ref[...]
    )


def _make_sc_aggregate(np_, chmax, chs0, chs1):
    stripe = np_ // _NS
    mesh = plsc.VectorSubcoreMesh(core_axis_name="c", subcore_axis_name="s")
    depth = 2

    @functools.partial(
        pl.kernel,
        mesh=mesh,
        out_type=jax.ShapeDtypeStruct((_NC, np_, _W), jnp.float32),
        scratch_types=[
            pltpu.VMEM((chmax, _CHUNK), jnp.int32),
            pltpu.VMEM((depth, _CHUNK), jnp.int32),
            pltpu.VMEM((depth, _CHUNK), jnp.int32),
            [pltpu.VMEM((_CHUNK, _W), jnp.float32) for _ in range(depth)],
            pltpu.VMEM_SHARED((np_, _W), jnp.float32),
            [pltpu.SemaphoreType.DMA for _ in range(depth)],
        ],
        compiler_params=pltpu.CompilerParams(use_tc_tiling_on_sc=False),
    )
    def sc_aggregate(yext, packed, zeros, out, packed_v, nbr_v, dst_v, rows, acc, sems):
        c = lax.axis_index("c")
        s = lax.axis_index("s")
        w = c * _NS + s
        row0 = s * stripe
        my_ch = jnp.where(c == 0, chs0, chs1)
        pltpu.sync_copy(zeros.at[pl.ds(row0, stripe)], acc.at[pl.ds(row0, stripe)])
        pltpu.sync_copy(packed.at[w], packed_v)
        plsc.subcore_barrier()

        def unpack(j, slot):
            for i in range(_CHUNK // 16):
                v = packed_v[j, pl.ds(16 * i, 16)]
                nbr_v[slot, pl.ds(16 * i, 16)] = lax.bitwise_and(v, 0xFFFF)
                dst_v[slot, pl.ds(16 * i, 16)] = lax.shift_right_logical(v, 16)

        def fire(j, slot):
            unpack(j, slot)
            pltpu.async_copy(yext.at[nbr_v.at[slot]], rows[slot], sems[slot])

        for k in range(depth - 1):
            fire(k, k)

        def step(j, slot):
            nxt = (slot + depth - 1) % depth

            @pl.when(j + depth - 1 < my_ch)
            def _():
                fire(j + depth - 1, nxt)

            @pl.when(j < my_ch)
            def _():
                pltpu.make_async_copy(
                    yext.at[nbr_v.at[slot]], rows[slot], sems[slot]
                ).wait()
                pltpu.sync_copy(rows[slot], acc.at[dst_v.at[slot]], add=True)

        def body(q, carry):
            j0 = depth * q
            for k in range(depth):
                step(j0 + k, k)
            return carry

        lax.fori_loop(0, chmax // depth, body, 0)
        plsc.subcore_barrier()
        pltpu.sync_copy(
            acc.at[pl.ds(row0, stripe)], out.at[c, pl.ds(row0, stripe)]
        )

    return sc_aggregate


def kernel(inputs, edges, edge_weights, W1, b1, W2, b2):
    del edge_weights
    _, n, d = inputs.shape
    h = W1.shape[1]
    e = edges.shape[1]

    np_ = ((n + _NS * 8 - 1) // (_NS * 8)) * (_NS * 8)
    frac0 = 0.62
    ct = -(-e // _CHUNK)
    chs0 = max(2, (int(ct * frac0) // (_NS * 2)) * 2)
    chs1 = -(-(ct - _NS * chs0) // _NS)
    chs1 += chs1 % 2
    chmax = max(chs0, chs1)
    ep = _NS * (chs0 + chs1) * _CHUNK

    x = inputs[0]
    xp = jnp.pad(x, ((0, np_ - n), (0, 0)))
    nbr = jnp.pad(edges[1], (0, ep - e))
    dst = jnp.pad(edges[0], (0, ep - e), constant_values=n)
    packed_flat = nbr | (dst << 16)
    pad_val = n << 16
    split = _NS * chs0 * _CHUNK
    part0 = packed_flat[:split].reshape(_NS, chs0, _CHUNK)
    part1 = packed_flat[split:].reshape(_NS, chs1, _CHUNK)
    part0 = jnp.pad(part0, ((0, 0), (0, chmax - chs0), (0, 0)),
                    constant_values=pad_val)
    part1 = jnp.pad(part1, ((0, 0), (0, chmax - chs1), (0, 0)),
                    constant_values=pad_val)
    packed = jnp.concatenate([part0, part1], axis=0)
    zeros = jnp.zeros((np_, _W), jnp.float32)

    yext = pl.pallas_call(
        _tc_message_ffn,
        out_shape=jax.ShapeDtypeStruct((np_, _W), jnp.float32),
    )(xp, W1, b1.reshape(1, h))

    partials = _make_sc_aggregate(np_, chmax, chs0, chs1)(yext, packed, zeros)

    out = pl.pallas_call(
        _tc_update_ffn,
        out_shape=jax.ShapeDtypeStruct((np_, h), jnp.float32),
    )(xp, partials, W2[:d], W2[d:], b2.reshape(1, h))

    return out[:n][None]
